# Initial kernel scaffold; baseline (speedup 1.0000x reference)
import jax
import jax.numpy as jnp
from jax.experimental import pallas as pl


def kernel(y, A, W, A_pinv, c1, c2, c3, parent, depth):
    raise NotImplementedError("write your pallas kernel here")


# speedup vs baseline: 22.2886x; 22.2886x over previous
import jax
import jax.numpy as jnp
from jax.experimental import pallas as pl
from jax.experimental.pallas import tpu as pltpu

M, N, B = 512, 2047, 64
NP = 2048
NUM_LAYERS = 16
RHO = 0.5
MAX_DEPTH = 10


def _lista_kernel(y_ref, at_ref, apt_ref, w_ref, dw_ref, anc_ref, sc_ref,
                  out_ref, x_ref, xp_ref):
    y = y_ref[...]
    A_T = at_ref[...]
    Apinv_T = apt_ref[...]
    W = w_ref[...]
    dw = dw_ref[...]
    c1a = sc_ref[0, 0]
    sigc2 = sc_ref[0, 1]

    lane = jax.lax.broadcasted_iota(jnp.int32, (1, NP), 1)
    valid = lane < N

    Apinv_y = jnp.dot(y, Apinv_T, preferred_element_type=jnp.float32)
    y_l1 = jnp.clip(jnp.sum(jnp.abs(Apinv_y), axis=-1, keepdims=True),
                    1e-12, None)

    x_ref[...] = jnp.zeros((B, NP), jnp.float32)
    xp_ref[...] = jnp.zeros((B, NP), jnp.float32)

    def layer(k, carry):
        x = x_ref[...]
        x_prev = xp_ref[...]
        residual = y - jnp.dot(x, A_T, preferred_element_type=jnp.float32)
        Apinv_res = jnp.dot(residual, Apinv_T,
                            preferred_element_type=jnp.float32)
        res_l1 = jnp.clip(jnp.sum(jnp.abs(Apinv_res), axis=-1, keepdims=True),
                          1e-12, None)
        residual_ratio = jnp.clip(res_l1 / y_l1, 0.0, 1.0)
        theta = c1a * residual_ratio
        active = jnp.sum((jnp.abs(x) > 1e-6).astype(jnp.float32),
                         axis=-1, keepdims=True)
        beta = sigc2 * (active / float(N))
        layer_progress = (k.astype(jnp.float32) + 1.0) * (1.0 / NUM_LAYERS)
        ratio = jnp.clip(y_l1 / res_l1, 1.0, None)
        log_ratio = jnp.clip(jnp.log(ratio), 0.0, None)
        signal_estimate = jax.nn.sigmoid(log_ratio - 1.0)
        c3a = sc_ref[0, 2]
        K_tree = jnp.clip(c3a * float(N) *
                          jnp.maximum(signal_estimate, layer_progress),
                          1.0, float(N) * 0.6)
        z = x + beta * (x - x_prev)
        residual2 = y - jnp.dot(z, A_T, preferred_element_type=jnp.float32)
        u = z + jnp.dot(residual2, W, preferred_element_type=jnp.float32)
        theta_s = jnp.mean(theta)
        Kv = jnp.maximum(jnp.floor(jnp.mean(K_tree)), 1.0).astype(jnp.int32)

        s = jnp.abs(u) * dw
        s_bits = jax.lax.bitcast_convert_type(s, jnp.int32)
        s_bits = jnp.where(valid, s_bits, -1)

        def cnt_ge(t):
            return jnp.sum((s_bits >= t).astype(jnp.int32),
                           axis=-1, keepdims=True)

        lo0 = jnp.zeros((B, 1), jnp.int32)
        hi0 = jnp.full((B, 1), 0x7F800001, jnp.int32)

        def bis(_, lh):
            lo, hi = lh
            mid = lo + (hi - lo) // 2
            ge = cnt_ge(mid) >= Kv
            return jnp.where(ge, mid, lo), jnp.where(ge, hi, mid)

        v_bits, _ = jax.lax.fori_loop(0, 31, bis, (lo0, hi0))
        gt = s_bits > v_bits
        eq = s_bits == v_bits
        c_gt = jnp.sum(gt.astype(jnp.int32), axis=-1, keepdims=True)
        need = Kv - c_gt

        def cnt_eq_le(t):
            return jnp.sum((eq & (lane <= t)).astype(jnp.int32),
                           axis=-1, keepdims=True)

        lo2 = jnp.full((B, 1), -1, jnp.int32)
        hi2 = jnp.full((B, 1), N - 1, jnp.int32)

        def bis2(_, lh):
            lo, hi = lh
            mid = lo + (hi - lo) // 2
            ge = cnt_eq_le(mid) >= need
            return jnp.where(ge, lo, mid), jnp.where(ge, mid, hi)

        _, J = jax.lax.fori_loop(0, 11, bis2, (lo2, hi2))
        mask = (gt | (eq & (lane <= J))).astype(jnp.bfloat16)

        closed = jnp.dot(mask, anc_ref[...],
                         preferred_element_type=jnp.float32)
        maskf = (closed > 0.5).astype(jnp.float32)

        x_new = jnp.sign(u) * jnp.maximum(jnp.abs(u) - theta_s, 0.0) * maskf
        x_new = jnp.where(valid, x_new, 0.0)
        xp_ref[...] = x
        x_ref[...] = x_new
        return carry

    jax.lax.fori_loop(0, NUM_LAYERS, layer, 0)
    out_ref[...] = x_ref[...]


def _pad_cols(a, np_):
    return jnp.pad(a, ((0, 0), (0, np_ - a.shape[1])))


def kernel(y, A, W, A_pinv, c1, c2, c3, parent, depth):
    A_T = jnp.pad(A.T, ((0, NP - N), (0, 0)))
    Apinv_T = _pad_cols(A_pinv.T, NP)
    Wp = _pad_cols(W, NP)
    dw = _pad_cols((RHO ** depth.astype(jnp.float32))[None, :], NP)

    cur = jnp.arange(N, dtype=jnp.int32)
    aa = jnp.arange(N, dtype=jnp.int32)[None, :]
    anc = jnp.zeros((N, N), jnp.bool_)
    for _ in range(MAX_DEPTH + 1):
        anc = anc | (cur[:, None] == aa)
        cur = parent[cur]
    anc_bf = jnp.pad(anc.astype(jnp.bfloat16),
                     ((0, NP - N), (0, NP - N)))

    sc = jnp.stack([jnp.abs(c1[0]), jax.nn.sigmoid(c2[0]),
                    jnp.abs(c3[0]), jnp.float32(0.0)]).reshape(1, 4)

    out = pl.pallas_call(
        _lista_kernel,
        out_shape=jax.ShapeDtypeStruct((B, NP), jnp.float32),
        scratch_shapes=[pltpu.VMEM((B, NP), jnp.float32),
                        pltpu.VMEM((B, NP), jnp.float32)],
    )(y, A_T, Apinv_T, Wp, dw, anc_bf, sc)
    return out[:, :N]


# speedup vs baseline: 23.9417x; 1.0742x over previous
import jax
import jax.numpy as jnp
from jax.experimental import pallas as pl
from jax.experimental.pallas import tpu as pltpu

M, N, B = 512, 2047, 64
NP = 2048
NUM_LAYERS = 16
RHO = 0.5
MAX_DEPTH = 10


def _lista_kernel(y_ref, at_ref, apt_ref, w_ref, dw_ref, anc_ref, sc_ref,
                  out_ref, x_ref, xp_ref, ax_ref, axp_ref):
    y = y_ref[...]
    A_T = at_ref[...]
    Apinv_T = apt_ref[...]
    W = w_ref[...]
    dw = dw_ref[...]
    c1a = sc_ref[0, 0]
    sigc2 = sc_ref[0, 1]

    lane = jax.lax.broadcasted_iota(jnp.int32, (1, NP), 1)
    valid = lane < N

    Apinv_y = jnp.dot(y, Apinv_T, preferred_element_type=jnp.float32)
    y_l1 = jnp.clip(jnp.sum(jnp.abs(Apinv_y), axis=-1, keepdims=True),
                    1e-12, None)

    x_ref[...] = jnp.zeros((B, NP), jnp.float32)
    xp_ref[...] = jnp.zeros((B, NP), jnp.float32)
    ax_ref[...] = jnp.zeros((B, M), jnp.float32)
    axp_ref[...] = jnp.zeros((B, M), jnp.float32)

    def layer(k, carry):
        x = x_ref[...]
        x_prev = xp_ref[...]
        Ax = ax_ref[...]
        Ax_prev = axp_ref[...]
        residual = y - Ax
        Apinv_res = jnp.dot(residual, Apinv_T,
                            preferred_element_type=jnp.float32)
        res_l1 = jnp.clip(jnp.sum(jnp.abs(Apinv_res), axis=-1, keepdims=True),
                          1e-12, None)
        residual_ratio = jnp.clip(res_l1 / y_l1, 0.0, 1.0)
        theta = c1a * residual_ratio
        active = jnp.sum((jnp.abs(x) > 1e-6).astype(jnp.float32),
                         axis=-1, keepdims=True)
        beta = sigc2 * (active / float(N))
        layer_progress = (k.astype(jnp.float32) + 1.0) * (1.0 / NUM_LAYERS)
        ratio = jnp.clip(y_l1 / res_l1, 1.0, None)
        log_ratio = jnp.clip(jnp.log(ratio), 0.0, None)
        signal_estimate = jax.nn.sigmoid(log_ratio - 1.0)
        c3a = sc_ref[0, 2]
        K_tree = jnp.clip(c3a * float(N) *
                          jnp.maximum(signal_estimate, layer_progress),
                          1.0, float(N) * 0.6)
        z = x + beta * (x - x_prev)
        residual2 = residual - beta * (Ax - Ax_prev)
        u = z + jnp.dot(residual2, W, preferred_element_type=jnp.float32)
        theta_s = jnp.mean(theta)
        Kv = jnp.maximum(jnp.floor(jnp.mean(K_tree)), 1.0).astype(jnp.int32)

        s = jnp.abs(u) * dw
        s_bits = jax.lax.bitcast_convert_type(s, jnp.int32)
        s_bits = jnp.where(valid, s_bits, -1)

        def cnt_ge(t):
            return jnp.sum((s_bits >= t).astype(jnp.int32),
                           axis=-1, keepdims=True)

        p = jnp.where(cnt_ge(jnp.full((B, 1), 1 << 30, jnp.int32)) >= Kv,
                      jnp.full((B, 1), 1 << 30, jnp.int32),
                      jnp.zeros((B, 1), jnp.int32))
        for shift in range(27, -1, -3):
            d = (cnt_ge(p + (1 << shift)) >= Kv).astype(jnp.int32)
            for k in range(2, 8):
                d = d + (cnt_ge(p + (k << shift)) >= Kv).astype(jnp.int32)
            p = p + (d << shift)
        v_bits = p
        gt = s_bits > v_bits
        eq = s_bits == v_bits
        c_gt = jnp.sum(gt.astype(jnp.int32), axis=-1, keepdims=True)
        need = Kv - c_gt

        def cnt_eq_le(t):
            return jnp.sum((eq & (lane <= t)).astype(jnp.int32),
                           axis=-1, keepdims=True)

        q = jnp.zeros((B, 1), jnp.int32)
        for shift in range(9, -1, -3):
            low1 = (1 << shift) - 1
            d = (cnt_eq_le(q + low1) < need).astype(jnp.int32)
            for k in range(1, 7):
                d = d + (cnt_eq_le(q + (k << shift) + low1) <
                         need).astype(jnp.int32)
            q = q + (d << shift)
        J = q
        mask = (gt | (eq & (lane <= J))).astype(jnp.bfloat16)

        closed = jnp.dot(mask, anc_ref[...],
                         preferred_element_type=jnp.float32)
        maskf = (closed > 0.5).astype(jnp.float32)

        x_new = jnp.sign(u) * jnp.maximum(jnp.abs(u) - theta_s, 0.0) * maskf
        x_new = jnp.where(valid, x_new, 0.0)
        xp_ref[...] = x
        x_ref[...] = x_new
        axp_ref[...] = Ax
        ax_ref[...] = jnp.dot(x_new, A_T, preferred_element_type=jnp.float32)
        return carry

    jax.lax.fori_loop(0, NUM_LAYERS, layer, 0)
    out_ref[...] = x_ref[...]


def _pad_cols(a, np_):
    return jnp.pad(a, ((0, 0), (0, np_ - a.shape[1])))


def kernel(y, A, W, A_pinv, c1, c2, c3, parent, depth):
    A_T = jnp.pad(A.T, ((0, NP - N), (0, 0)))
    Apinv_T = _pad_cols(A_pinv.T, NP)
    Wp = _pad_cols(W, NP)
    dw = _pad_cols((RHO ** depth.astype(jnp.float32))[None, :], NP)

    cur = jnp.arange(N, dtype=jnp.int32)
    aa = jnp.arange(N, dtype=jnp.int32)[None, :]
    anc = jnp.zeros((N, N), jnp.bool_)
    for _ in range(MAX_DEPTH + 1):
        anc = anc | (cur[:, None] == aa)
        cur = parent[cur]
    anc_bf = jnp.pad(anc.astype(jnp.bfloat16),
                     ((0, NP - N), (0, NP - N)))

    sc = jnp.stack([jnp.abs(c1[0]), jax.nn.sigmoid(c2[0]),
                    jnp.abs(c3[0]), jnp.float32(0.0)]).reshape(1, 4)

    out = pl.pallas_call(
        _lista_kernel,
        out_shape=jax.ShapeDtypeStruct((B, NP), jnp.float32),
        scratch_shapes=[pltpu.VMEM((B, NP), jnp.float32),
                        pltpu.VMEM((B, NP), jnp.float32),
                        pltpu.VMEM((B, M), jnp.float32),
                        pltpu.VMEM((B, M), jnp.float32)],
    )(y, A_T, Apinv_T, Wp, dw, anc_bf, sc)
    return out[:, :N]
